# baseline (speedup 1.0000x reference)
import functools

import jax
import jax.numpy as jnp
from jax import lax
from jax.experimental import pallas as pl
from jax.experimental.pallas import tpu as pltpu
from jax.experimental.pallas import tpu_sc as plsc

N = 16384
K = 8192
D = 256
TN = 512

NC = 2
NS = 16
NW = NC * NS
BPW = N // NW
CB = 64
NCH = BPW // CB
LANES = 16


WCH = 1024
NCHK = K // WCH


def _argmin_body(z_ref, w_ref, out_ref, wsq_ref):
    i = pl.program_id(0)

    @pl.when(i == 0)
    def _():
        w = w_ref[...]
        wsq_ref[...] = jnp.sum(w * w, axis=1)[None, :]

    z = z_ref[...]
    zsq = jnp.sum(z * z, axis=1, keepdims=True)
    z2 = z + z
    mm2 = lax.dot_general(z2, w_ref[...], (((1,), (1,)), ((), ())),
                          preferred_element_type=jnp.float32)
    d = (zsq + wsq_ref[...]) - mm2
    mins = jnp.min(d, axis=1, keepdims=True)
    ks = lax.broadcasted_iota(jnp.int32, d.shape, 1)
    cand = jnp.where(d == mins, ks, jnp.int32(K))
    out_ref[...] = jnp.min(cand, axis=1).reshape(1, 1, TN)


def _argmin_call(z_e, W):
    out = pl.pallas_call(
        _argmin_body,
        grid=(N // TN,),
        in_specs=[
            pl.BlockSpec((TN, D), lambda i: (i, 0)),
            pl.BlockSpec((K, D), lambda i: (0, 0)),
        ],
        out_specs=pl.BlockSpec((1, 1, TN), lambda i: (i, 0, 0)),
        out_shape=jax.ShapeDtypeStruct((N // TN, 1, TN), jnp.int32),
        scratch_shapes=[pltpu.VMEM((1, K), jnp.float32)],
    )(z_e, W)
    return out.reshape(N)


@functools.cache
def _build_gather_loss():
    mesh = plsc.VectorSubcoreMesh(core_axis_name="c", subcore_axis_name="s",
                                  num_cores=NC, num_subcores=NS)

    @functools.partial(
        pl.kernel,
        out_type=(
            jax.ShapeDtypeStruct((N, D), jnp.float32),
            jax.ShapeDtypeStruct((NW, LANES), jnp.float32),
        ),
        mesh=mesh,
        scratch_types=[
            pltpu.VMEM((CB,), jnp.int32),
            pltpu.VMEM((CB, D), jnp.float32),
            pltpu.VMEM((CB, D), jnp.float32),
            pltpu.VMEM((LANES,), jnp.float32),
            pltpu.SemaphoreType.DMA,
        ],
    )
    def _gather_loss(w_hbm, idx_hbm, ze_hbm, zq_hbm, loss_hbm,
                     idx_v, rows_v, ze_v, acc_v, sem):
        wid = lax.axis_index("s") * NC + lax.axis_index("c")

        def chunk(ci, acc):
            base = wid * BPW + ci * CB
            pltpu.sync_copy(idx_hbm.at[pl.ds(base, CB)], idx_v)
            cp = pltpu.async_copy(w_hbm.at[idx_v], rows_v, sem)
            pltpu.sync_copy(ze_hbm.at[pl.ds(base, CB)], ze_v)
            cp.wait()

            def row(r, a):
                for c in range(D // LANES):
                    sl = pl.ds(c * LANES, LANES)
                    ze = ze_v[r, sl]
                    dlt = rows_v[r, sl] - ze
                    a = a + dlt * dlt
                    rows_v[r, sl] = ze + dlt
                return a

            acc = lax.fori_loop(0, CB, row, acc)
            pltpu.sync_copy(rows_v, zq_hbm.at[pl.ds(base, CB)])
            return acc

        acc = lax.fori_loop(0, NCH, chunk, jnp.zeros((LANES,), jnp.float32))
        acc_v[...] = acc
        pltpu.sync_copy(acc_v, loss_hbm.at[wid])

    return _gather_loss


def kernel(z_e, W):
    indices = _argmin_call(z_e, W)
    z_q_st, loss_parts = _build_gather_loss()(W, indices, z_e)
    m = jnp.sum(loss_parts) / jnp.float32(N * D)
    vq_loss = m + 0.25 * m
    return (z_q_st, vq_loss, indices)


# speedup vs baseline: 1.1467x; 1.1467x over previous
import functools

import jax
import jax.numpy as jnp
from jax import lax
from jax.experimental import pallas as pl
from jax.experimental.pallas import tpu as pltpu
from jax.experimental.pallas import tpu_sc as plsc

N = 16384
K = 8192
D = 256
TN = 512

NC = 2
NS = 16
NW = NC * NS
BPW = N // NW
CB = 64
NCH = BPW // CB
LANES = 16


WCH = 1024
NCHK = K // WCH


def _argmin_body(z_ref, w_ref, out_ref, wsq_ref):
    i = pl.program_id(0)

    @pl.when(i == 0)
    def _():
        w = w_ref[...]
        wsq_ref[...] = jnp.sum(w * w, axis=1)[None, :]

    z = z_ref[...]
    zsq = jnp.sum(z * z, axis=1, keepdims=True)
    mm = lax.dot_general(z, w_ref[...], (((1,), (1,)), ((), ())),
                         preferred_element_type=jnp.float32)
    d = (zsq + wsq_ref[...]) - 2.0 * mm
    mins = jnp.min(d, axis=1, keepdims=True)
    ks = lax.broadcasted_iota(jnp.int32, d.shape, 1)
    cand = jnp.where(d == mins, ks, jnp.int32(K))
    out_ref[...] = jnp.min(cand, axis=1).reshape(1, 1, TN)


def _argmin_call(z_e, W):
    out = pl.pallas_call(
        _argmin_body,
        grid=(N // TN,),
        in_specs=[
            pl.BlockSpec((TN, D), lambda i: (i, 0)),
            pl.BlockSpec((K, D), lambda i: (0, 0)),
        ],
        out_specs=pl.BlockSpec((1, 1, TN), lambda i: (i, 0, 0)),
        out_shape=jax.ShapeDtypeStruct((N // TN, 1, TN), jnp.int32),
        scratch_shapes=[pltpu.VMEM((1, K), jnp.float32)],
    )(z_e, W)
    return out.reshape(N)


@functools.cache
def _build_gather_loss():
    mesh = plsc.VectorSubcoreMesh(core_axis_name="c", subcore_axis_name="s",
                                  num_cores=NC, num_subcores=NS)

    @functools.partial(
        pl.kernel,
        out_type=(
            jax.ShapeDtypeStruct((N, D), jnp.float32),
            jax.ShapeDtypeStruct((NW, LANES), jnp.float32),
        ),
        mesh=mesh,
        scratch_types=[
            pltpu.VMEM((CB,), jnp.int32),
            pltpu.VMEM((CB, D), jnp.float32),
            pltpu.VMEM((CB, D), jnp.float32),
            pltpu.VMEM((LANES,), jnp.float32),
            pltpu.SemaphoreType.DMA,
        ],
    )
    def _gather_loss(w_hbm, idx_hbm, ze_hbm, zq_hbm, loss_hbm,
                     idx_v, rows_v, ze_v, acc_v, sem):
        wid = lax.axis_index("s") * NC + lax.axis_index("c")

        def chunk(ci, acc):
            base = wid * BPW + ci * CB
            pltpu.sync_copy(idx_hbm.at[pl.ds(base, CB)], idx_v)
            cp = pltpu.async_copy(w_hbm.at[idx_v], rows_v, sem)
            pltpu.sync_copy(ze_hbm.at[pl.ds(base, CB)], ze_v)
            cp.wait()

            def row(r, a):
                for c in range(D // LANES):
                    sl = pl.ds(c * LANES, LANES)
                    ze = ze_v[r, sl]
                    dlt = rows_v[r, sl] - ze
                    a = a + dlt * dlt
                    rows_v[r, sl] = ze + dlt
                return a

            acc = lax.fori_loop(0, CB, row, acc)
            pltpu.sync_copy(rows_v, zq_hbm.at[pl.ds(base, CB)])
            return acc

        acc = lax.fori_loop(0, NCH, chunk, jnp.zeros((LANES,), jnp.float32))
        acc_v[...] = acc
        pltpu.sync_copy(acc_v, loss_hbm.at[wid])

    return _gather_loss


def kernel(z_e, W):
    indices = _argmin_call(z_e, W)
    z_q_st, loss_parts = _build_gather_loss()(W, indices, z_e)
    m = jnp.sum(loss_parts) / jnp.float32(N * D)
    vq_loss = m + 0.25 * m
    return (z_q_st, vq_loss, indices)


# speedup vs baseline: 1.2426x; 1.0836x over previous
import functools

import jax
import jax.numpy as jnp
from jax import lax
from jax.experimental import pallas as pl
from jax.experimental.pallas import tpu as pltpu
from jax.experimental.pallas import tpu_sc as plsc

N = 16384
K = 8192
D = 256
TN = 512

NC = 2
NS = 16
NW = NC * NS
BPW = N // NW
CB = 64
NCH = BPW // CB
LANES = 16


WCH = 1024
NCHK = K // WCH


def _argmin_body(z_ref, w_ref, out_ref, wsq_ref):
    i = pl.program_id(0)

    @pl.when(i == 0)
    def _():
        w = w_ref[...]
        wsq_ref[...] = jnp.sum(w * w, axis=1)[None, :]

    z = z_ref[...]
    zsq = jnp.sum(z * z, axis=1, keepdims=True)
    mm = lax.dot_general(z, w_ref[...], (((1,), (1,)), ((), ())),
                         preferred_element_type=jnp.float32)
    d = (zsq + wsq_ref[...]) - 2.0 * mm
    out_ref[...] = jnp.argmin(d, axis=1).astype(jnp.int32).reshape(1, 1, TN)


def _argmin_call(z_e, W):
    out = pl.pallas_call(
        _argmin_body,
        grid=(N // TN,),
        in_specs=[
            pl.BlockSpec((TN, D), lambda i: (i, 0)),
            pl.BlockSpec((K, D), lambda i: (0, 0)),
        ],
        out_specs=pl.BlockSpec((1, 1, TN), lambda i: (i, 0, 0)),
        out_shape=jax.ShapeDtypeStruct((N // TN, 1, TN), jnp.int32),
        scratch_shapes=[pltpu.VMEM((1, K), jnp.float32)],
    )(z_e, W)
    return out.reshape(N)


@functools.cache
def _build_gather_loss():
    mesh = plsc.VectorSubcoreMesh(core_axis_name="c", subcore_axis_name="s",
                                  num_cores=NC, num_subcores=NS)

    @functools.partial(
        pl.kernel,
        out_type=(
            jax.ShapeDtypeStruct((N, D), jnp.float32),
            jax.ShapeDtypeStruct((NW, LANES), jnp.float32),
        ),
        mesh=mesh,
        scratch_types=[
            pltpu.VMEM((CB,), jnp.int32),
            pltpu.VMEM((CB, D), jnp.float32),
            pltpu.VMEM((CB, D), jnp.float32),
            pltpu.VMEM((LANES,), jnp.float32),
            pltpu.SemaphoreType.DMA,
        ],
    )
    def _gather_loss(w_hbm, idx_hbm, ze_hbm, zq_hbm, loss_hbm,
                     idx_v, rows_v, ze_v, acc_v, sem):
        wid = lax.axis_index("s") * NC + lax.axis_index("c")

        def chunk(ci, acc):
            base = wid * BPW + ci * CB
            pltpu.sync_copy(idx_hbm.at[pl.ds(base, CB)], idx_v)
            cp = pltpu.async_copy(w_hbm.at[idx_v], rows_v, sem)
            pltpu.sync_copy(ze_hbm.at[pl.ds(base, CB)], ze_v)
            cp.wait()

            def row(r, a):
                for c in range(D // LANES):
                    sl = pl.ds(c * LANES, LANES)
                    ze = ze_v[r, sl]
                    dlt = rows_v[r, sl] - ze
                    a = a + dlt * dlt
                    rows_v[r, sl] = ze + dlt
                return a

            acc = lax.fori_loop(0, CB, row, acc)
            pltpu.sync_copy(rows_v, zq_hbm.at[pl.ds(base, CB)])
            return acc

        acc = lax.fori_loop(0, NCH, chunk, jnp.zeros((LANES,), jnp.float32))
        acc_v[...] = acc
        pltpu.sync_copy(acc_v, loss_hbm.at[wid])

    return _gather_loss


def kernel(z_e, W):
    indices = _argmin_call(z_e, W)
    z_q_st, loss_parts = _build_gather_loss()(W, indices, z_e)
    m = jnp.sum(loss_parts) / jnp.float32(N * D)
    vq_loss = m + 0.25 * m
    return (z_q_st, vq_loss, indices)


# speedup vs baseline: 1.2819x; 1.0316x over previous
import functools

import jax
import jax.numpy as jnp
from jax import lax
from jax.experimental import pallas as pl
from jax.experimental.pallas import tpu as pltpu
from jax.experimental.pallas import tpu_sc as plsc

N = 16384
K = 8192
D = 256
TN = 512

NC = 2
NS = 16
NW = NC * NS
BPW = N // NW
CB = 64
NCH = BPW // CB
LANES = 16


WCH = 1024
NCHK = K // WCH


def _argmin_body(z_ref, w_ref, out_ref, wsq_ref, iota_ref):
    i = pl.program_id(0)

    @pl.when(i == 0)
    def _():
        w = w_ref[...]
        wsq_ref[...] = jnp.sum(w * w, axis=1)[None, :]
        iota_ref[...] = lax.broadcasted_iota(jnp.int32, (1, K), 1).astype(jnp.float32)

    z = z_ref[...]
    zsq = jnp.sum(z * z, axis=1, keepdims=True)
    mm = lax.dot_general(z, w_ref[...], (((1,), (1,)), ((), ())),
                         preferred_element_type=jnp.float32)
    d = (zsq + wsq_ref[...]) - 2.0 * mm
    mins = jnp.min(d, axis=1, keepdims=True)
    cand = jnp.where(d == mins, iota_ref[...], jnp.float32(K))
    out_ref[...] = jnp.min(cand, axis=1).astype(jnp.int32).reshape(1, 1, TN)


def _argmin_call(z_e, W):
    out = pl.pallas_call(
        _argmin_body,
        grid=(N // TN,),
        in_specs=[
            pl.BlockSpec((TN, D), lambda i: (i, 0)),
            pl.BlockSpec((K, D), lambda i: (0, 0)),
        ],
        out_specs=pl.BlockSpec((1, 1, TN), lambda i: (i, 0, 0)),
        out_shape=jax.ShapeDtypeStruct((N // TN, 1, TN), jnp.int32),
        scratch_shapes=[pltpu.VMEM((1, K), jnp.float32),
                        pltpu.VMEM((1, K), jnp.float32)],
    )(z_e, W)
    return out.reshape(N)


@functools.cache
def _build_gather_loss():
    mesh = plsc.VectorSubcoreMesh(core_axis_name="c", subcore_axis_name="s",
                                  num_cores=NC, num_subcores=NS)

    @functools.partial(
        pl.kernel,
        out_type=(
            jax.ShapeDtypeStruct((N, D), jnp.float32),
            jax.ShapeDtypeStruct((NW, LANES), jnp.float32),
        ),
        mesh=mesh,
        scratch_types=[
            pltpu.VMEM((CB,), jnp.int32),
            pltpu.VMEM((CB, D), jnp.float32),
            pltpu.VMEM((CB, D), jnp.float32),
            pltpu.VMEM((LANES,), jnp.float32),
            pltpu.SemaphoreType.DMA,
        ],
    )
    def _gather_loss(w_hbm, idx_hbm, ze_hbm, zq_hbm, loss_hbm,
                     idx_v, rows_v, ze_v, acc_v, sem):
        wid = lax.axis_index("s") * NC + lax.axis_index("c")

        def chunk(ci, acc):
            base = wid * BPW + ci * CB
            pltpu.sync_copy(idx_hbm.at[pl.ds(base, CB)], idx_v)
            cp = pltpu.async_copy(w_hbm.at[idx_v], rows_v, sem)
            pltpu.sync_copy(ze_hbm.at[pl.ds(base, CB)], ze_v)
            cp.wait()

            def row(r, a):
                for c in range(D // LANES):
                    sl = pl.ds(c * LANES, LANES)
                    ze = ze_v[r, sl]
                    dlt = rows_v[r, sl] - ze
                    a = a + dlt * dlt
                    rows_v[r, sl] = ze + dlt
                return a

            acc = lax.fori_loop(0, CB, row, acc)
            pltpu.sync_copy(rows_v, zq_hbm.at[pl.ds(base, CB)])
            return acc

        acc = lax.fori_loop(0, NCH, chunk, jnp.zeros((LANES,), jnp.float32))
        acc_v[...] = acc
        pltpu.sync_copy(acc_v, loss_hbm.at[wid])

    return _gather_loss


def kernel(z_e, W):
    indices = _argmin_call(z_e, W)
    z_q_st, loss_parts = _build_gather_loss()(W, indices, z_e)
    m = jnp.sum(loss_parts) / jnp.float32(N * D)
    vq_loss = m + 0.25 * m
    return (z_q_st, vq_loss, indices)
